# baseline (speedup 1.0000x reference)
import functools

import jax
import jax.numpy as jnp
from jax import lax
from jax.experimental import pallas as pl
from jax.experimental.pallas import tpu as pltpu
from jax.experimental.pallas import tpu_sc as plsc

_B = 4096
_K = 128
_T = 4096


def _sc_gather(stu_id, exe_id, student_emb, k_difficulty, e_pad):
    info = plsc.get_sparse_core_info()
    nc, ns = info.num_cores, info.num_subcores
    nw = nc * ns
    bpw = _B // nw
    mesh = plsc.VectorSubcoreMesh(core_axis_name="c", subcore_axis_name="s")

    @functools.partial(
        pl.kernel,
        mesh=mesh,
        out_type=[
            jax.ShapeDtypeStruct((_B, _K), jnp.float32),
            jax.ShapeDtypeStruct((_B, _K), jnp.float32),
            jax.ShapeDtypeStruct((_B, _K), jnp.float32),
        ],
        scratch_types=[
            pltpu.VMEM((bpw,), jnp.int32),
            pltpu.VMEM((bpw,), jnp.int32),
            pltpu.VMEM((bpw,), jnp.int32),
            pltpu.VMEM((bpw, _K), jnp.float32),
            pltpu.VMEM((bpw, _K), jnp.float32),
            pltpu.VMEM((bpw, _K), jnp.float32),
            pltpu.SemaphoreType.DMA,
        ],
    )
    def k(stu_hbm, exe_hbm, se_hbm, kd_hbm, ep_hbm, out_se, out_kd, out_ew,
          stu_v, exe_v, erow_v, se_v, kd_v, ewin_v, sem):
        wid = lax.axis_index("s") * nc + lax.axis_index("c")
        base = wid * bpw
        pltpu.sync_copy(stu_hbm.at[pl.ds(base, bpw)], stu_v)
        pltpu.sync_copy(exe_hbm.at[pl.ds(base, bpw)], exe_v)
        for c in range(bpw // 16):
            erow_v[pl.ds(c * 16, 16)] = lax.shift_right_logical(
                exe_v[pl.ds(c * 16, 16)], 7)
        c1 = pltpu.async_copy(se_hbm.at[stu_v], se_v, sem)
        c2 = pltpu.async_copy(kd_hbm.at[exe_v], kd_v, sem)
        c3 = pltpu.async_copy(ep_hbm.at[erow_v], ewin_v, sem)
        c1.wait()
        c2.wait()
        c3.wait()
        pltpu.sync_copy(se_v, out_se.at[pl.ds(base, bpw)])
        pltpu.sync_copy(kd_v, out_kd.at[pl.ds(base, bpw)])
        pltpu.sync_copy(ewin_v, out_ew.at[pl.ds(base, bpw)])

    return k(stu_id, exe_id, student_emb, k_difficulty, e_pad)


def _tc_proj_body(tt_ref, wkc_ref, bkc_ref, ikp_ref, ikp2_ref):
    x = lax.dot_general(tt_ref[...], wkc_ref[...], (((1,), (1,)), ((), ())),
                        preferred_element_type=jnp.float32)
    x = x + bkc_ref[...]
    ikp = ikp_ref[...]
    m1 = jnp.max(x, axis=1, keepdims=True)
    x2 = jnp.where(x == m1, -jnp.inf, x)
    m2 = jnp.max(x2, axis=1, keepdims=True)
    merged = jnp.logical_or(x >= m2, ikp != 0.0)
    s = x + ikp
    norm = jnp.maximum(jnp.sqrt(jnp.sum(s * s, axis=1, keepdims=True)), 1e-12)
    ikp2_ref[...] = jnp.where(merged, s / norm, 0.0)


def _tc_proj(tt, wkc, bkc, ikp):
    bb = 512
    grid = _B // bb
    return pl.pallas_call(
        _tc_proj_body,
        grid=(grid,),
        in_specs=[
            pl.BlockSpec((bb, _T), lambda i: (i, 0)),
            pl.BlockSpec((_K, _T), lambda i: (0, 0)),
            pl.BlockSpec((1, _K), lambda i: (0, 0)),
            pl.BlockSpec((bb, _K), lambda i: (i, 0)),
        ],
        out_specs=pl.BlockSpec((bb, _K), lambda i: (i, 0)),
        out_shape=jax.ShapeDtypeStruct((_B, _K), jnp.float32),
    )(tt, wkc, bkc.reshape(1, _K), ikp)


def _tc_mlp_body(ikp2_ref, se_ref, kd_ref, ew_ref, exe_ref,
                 w1_ref, b1_ref, w2_ref, b2_ref, w3_ref, b3_ref, out_ref):
    col = lax.bitwise_and(exe_ref[...], 127)
    lane = lax.broadcasted_iota(jnp.int32, ew_ref.shape, 1)
    ed = jnp.sum(jnp.where(lane == col, ew_ref[...], 0.0), axis=1,
                 keepdims=True)
    stat = jax.nn.sigmoid(se_ref[...])
    esig = jax.nn.sigmoid(ed)
    input_x = esig * (stat - kd_ref[...]) * ikp2_ref[...]
    h = jax.nn.sigmoid(
        lax.dot_general(input_x, jnp.abs(w1_ref[...]), (((1,), (1,)), ((), ())),
                        preferred_element_type=jnp.float32) + b1_ref[...])
    h = jax.nn.sigmoid(
        lax.dot_general(h, jnp.abs(w2_ref[...]), (((1,), (1,)), ((), ())),
                        preferred_element_type=jnp.float32) + b2_ref[...])
    o = jax.nn.sigmoid(
        jnp.sum(h * jnp.abs(w3_ref[...]), axis=1, keepdims=True)
        + b3_ref[0])
    out_ref[...] = o


def _tc_mlp(ikp2, se, kd, ew, exe, w1, b1, w2, b2, w3, b3):
    bb = 512
    grid = _B // bb
    return pl.pallas_call(
        _tc_mlp_body,
        grid=(grid,),
        in_specs=[
            pl.BlockSpec((bb, _K), lambda i: (i, 0)),
            pl.BlockSpec((bb, _K), lambda i: (i, 0)),
            pl.BlockSpec((bb, _K), lambda i: (i, 0)),
            pl.BlockSpec((bb, _K), lambda i: (i, 0)),
            pl.BlockSpec((bb, 1), lambda i: (i, 0)),
            pl.BlockSpec((512, _K), lambda i: (0, 0)),
            pl.BlockSpec((1, 512), lambda i: (0, 0)),
            pl.BlockSpec((256, 512), lambda i: (0, 0)),
            pl.BlockSpec((1, 256), lambda i: (0, 0)),
            pl.BlockSpec((1, 256), lambda i: (0, 0)),
            pl.BlockSpec(memory_space=pltpu.SMEM),
        ],
        out_specs=pl.BlockSpec((bb, 1), lambda i: (i, 0)),
        out_shape=jax.ShapeDtypeStruct((_B, 1), jnp.float32),
    )(ikp2, se, kd, ew, exe.reshape(_B, 1),
      w1, b1.reshape(1, 512), w2, b2.reshape(1, 256), w3, b3)


def kernel(stu_id, input_exercise, input_knowledge_point, teacher_text,
           student_emb, k_difficulty, e_difficulty, Wkc, bkc,
           W1, b1, W2, b2, W3, b3):
    n_e = e_difficulty.shape[0]
    ew_rows = (n_e + _K - 1) // _K
    e_pad = jnp.concatenate(
        [e_difficulty.reshape(-1),
         jnp.zeros((ew_rows * _K - n_e,), jnp.float32)]).reshape(ew_rows, _K)
    se, kd, ew = _sc_gather(stu_id, input_exercise,
                            student_emb, k_difficulty, e_pad)
    ikp2 = _tc_proj(teacher_text, Wkc, bkc, input_knowledge_point)
    out = _tc_mlp(ikp2, se, kd, ew, input_exercise, W1, b1, W2, b2, W3, b3)
    return out.reshape(-1)


# speedup vs baseline: 1.0946x; 1.0946x over previous
import functools

import jax
import jax.numpy as jnp
from jax import lax
from jax.experimental import pallas as pl
from jax.experimental.pallas import tpu as pltpu
from jax.experimental.pallas import tpu_sc as plsc

_B = 4096
_K = 128
_T = 4096


def _sc_gather(stu_id, exe_id, student_emb, k_difficulty, e_pad):
    info = plsc.get_sparse_core_info()
    nc, ns = info.num_cores, info.num_subcores
    nw = nc * ns
    bpw = _B // nw
    mesh = plsc.VectorSubcoreMesh(core_axis_name="c", subcore_axis_name="s")

    @functools.partial(
        pl.kernel,
        mesh=mesh,
        out_type=[
            jax.ShapeDtypeStruct((_B, _K), jnp.float32),
            jax.ShapeDtypeStruct((_B, _K), jnp.float32),
            jax.ShapeDtypeStruct((_B, _K), jnp.float32),
        ],
        scratch_types=[
            pltpu.VMEM((bpw,), jnp.int32),
            pltpu.VMEM((bpw,), jnp.int32),
            pltpu.VMEM((bpw,), jnp.int32),
            pltpu.VMEM((bpw, _K), jnp.float32),
            pltpu.VMEM((bpw, _K), jnp.float32),
            pltpu.VMEM((bpw, _K), jnp.float32),
            pltpu.SemaphoreType.DMA,
        ],
    )
    def k(stu_hbm, exe_hbm, se_hbm, kd_hbm, ep_hbm, out_se, out_kd, out_ew,
          stu_v, exe_v, erow_v, se_v, kd_v, ewin_v, sem):
        wid = lax.axis_index("s") * nc + lax.axis_index("c")
        base = wid * bpw
        i1 = pltpu.async_copy(stu_hbm.at[pl.ds(base, bpw)], stu_v, sem)
        i2 = pltpu.async_copy(exe_hbm.at[pl.ds(base, bpw)], exe_v, sem)
        i2.wait()
        for c in range(bpw // 16):
            erow_v[pl.ds(c * 16, 16)] = lax.shift_right_logical(
                exe_v[pl.ds(c * 16, 16)], 7)
        c2 = pltpu.async_copy(kd_hbm.at[exe_v], kd_v, sem)
        c3 = pltpu.async_copy(ep_hbm.at[erow_v], ewin_v, sem)
        i1.wait()
        c1 = pltpu.async_copy(se_hbm.at[stu_v], se_v, sem)
        c2.wait()
        pltpu.sync_copy(kd_v, out_kd.at[pl.ds(base, bpw)])
        c3.wait()
        pltpu.sync_copy(ewin_v, out_ew.at[pl.ds(base, bpw)])
        c1.wait()
        pltpu.sync_copy(se_v, out_se.at[pl.ds(base, bpw)])

    return k(stu_id, exe_id, student_emb, k_difficulty, e_pad)


def _tc_body(tt_ref, wkc_ref, bkc_ref, ikp_ref, se_ref, kd_ref, ew_ref,
             exe_ref, w1_ref, b1_ref, w2_ref, b2_ref, w3_ref, b3_ref,
             out_ref):
    x = lax.dot_general(tt_ref[...], wkc_ref[...], (((1,), (1,)), ((), ())),
                        preferred_element_type=jnp.float32)
    x = x + bkc_ref[...]
    ikp = ikp_ref[...]
    col = lax.bitwise_and(exe_ref[...], 127)
    lane = lax.broadcasted_iota(jnp.int32, ew_ref.shape, 1)
    ed = jnp.sum(jnp.where(lane == col, ew_ref[...], 0.0), axis=1,
                 keepdims=True)
    m1 = jnp.max(x, axis=1, keepdims=True)
    x2 = jnp.where(x == m1, -jnp.inf, x)
    m2 = jnp.max(x2, axis=1, keepdims=True)
    merged = jnp.logical_or(x >= m2, ikp != 0.0)
    s = x + ikp
    norm = jnp.maximum(jnp.sqrt(jnp.sum(s * s, axis=1, keepdims=True)), 1e-12)
    ikp2 = jnp.where(merged, s / norm, 0.0)
    stat = jax.nn.sigmoid(se_ref[...])
    esig = jax.nn.sigmoid(ed)
    input_x = esig * (stat - kd_ref[...]) * ikp2
    h = jax.nn.sigmoid(
        lax.dot_general(input_x, jnp.abs(w1_ref[...]), (((1,), (1,)), ((), ())),
                        preferred_element_type=jnp.float32) + b1_ref[...])
    h = jax.nn.sigmoid(
        lax.dot_general(h, jnp.abs(w2_ref[...]), (((1,), (1,)), ((), ())),
                        preferred_element_type=jnp.float32) + b2_ref[...])
    o = jax.nn.sigmoid(
        jnp.sum(h * jnp.abs(w3_ref[...]), axis=1, keepdims=True)
        + b3_ref[0])
    out_ref[...] = o


def _tc_dense(tt, wkc, bkc, ikp, se, kd, ew, exe, w1, b1, w2, b2, w3, b3):
    bb = 512
    grid = _B // bb
    return pl.pallas_call(
        _tc_body,
        grid=(grid,),
        in_specs=[
            pl.BlockSpec((bb, _T), lambda i: (i, 0)),
            pl.BlockSpec((_K, _T), lambda i: (0, 0)),
            pl.BlockSpec((1, _K), lambda i: (0, 0)),
            pl.BlockSpec((bb, _K), lambda i: (i, 0)),
            pl.BlockSpec((bb, _K), lambda i: (i, 0)),
            pl.BlockSpec((bb, _K), lambda i: (i, 0)),
            pl.BlockSpec((bb, _K), lambda i: (i, 0)),
            pl.BlockSpec((bb, 1), lambda i: (i, 0)),
            pl.BlockSpec((512, _K), lambda i: (0, 0)),
            pl.BlockSpec((1, 512), lambda i: (0, 0)),
            pl.BlockSpec((256, 512), lambda i: (0, 0)),
            pl.BlockSpec((1, 256), lambda i: (0, 0)),
            pl.BlockSpec((1, 256), lambda i: (0, 0)),
            pl.BlockSpec(memory_space=pltpu.SMEM),
        ],
        out_specs=pl.BlockSpec((bb, 1), lambda i: (i, 0)),
        out_shape=jax.ShapeDtypeStruct((_B, 1), jnp.float32),
    )(tt, wkc, bkc.reshape(1, _K), ikp, se, kd, ew, exe.reshape(_B, 1),
      w1, b1.reshape(1, 512), w2, b2.reshape(1, 256), w3, b3)


def kernel(stu_id, input_exercise, input_knowledge_point, teacher_text,
           student_emb, k_difficulty, e_difficulty, Wkc, bkc,
           W1, b1, W2, b2, W3, b3):
    n_e = e_difficulty.shape[0]
    ew_rows = (n_e + _K - 1) // _K
    e_pad = jnp.concatenate(
        [e_difficulty.reshape(-1),
         jnp.zeros((ew_rows * _K - n_e,), jnp.float32)]).reshape(ew_rows, _K)
    se, kd, ew = _sc_gather(stu_id, input_exercise,
                            student_emb, k_difficulty, e_pad)
    out = _tc_dense(teacher_text, Wkc, bkc, input_knowledge_point,
                    se, kd, ew, input_exercise, W1, b1, W2, b2, W3, b3)
    return out.reshape(-1)


# speedup vs baseline: 1.1476x; 1.0484x over previous
import functools

import jax
import jax.numpy as jnp
from jax import lax
from jax.experimental import pallas as pl
from jax.experimental.pallas import tpu as pltpu
from jax.experimental.pallas import tpu_sc as plsc

_B = 4096
_K = 128
_T = 4096


def _sc_gather(stu_id, exe_id, student_emb, k_difficulty, e_pad):
    info = plsc.get_sparse_core_info()
    nc, ns = info.num_cores, info.num_subcores
    nw = nc * ns
    bpw = _B // nw
    mesh = plsc.VectorSubcoreMesh(core_axis_name="c", subcore_axis_name="s")

    @functools.partial(
        pl.kernel,
        mesh=mesh,
        out_type=[
            jax.ShapeDtypeStruct((_B, _K), jnp.float32),
            jax.ShapeDtypeStruct((_B, _K), jnp.float32),
            jax.ShapeDtypeStruct((_B, _K), jnp.float32),
        ],
        scratch_types=[
            pltpu.VMEM((bpw,), jnp.int32),
            pltpu.VMEM((bpw,), jnp.int32),
            pltpu.VMEM((bpw,), jnp.int32),
            pltpu.VMEM((bpw, _K), jnp.float32),
            pltpu.VMEM((bpw, _K), jnp.float32),
            pltpu.VMEM((bpw, _K), jnp.float32),
            pltpu.SemaphoreType.DMA,
        ],
    )
    def k(stu_hbm, exe_hbm, se_hbm, kd_hbm, ep_hbm, out_se, out_kd, out_ew,
          stu_v, exe_v, erow_v, se_v, kd_v, ewin_v, sem):
        wid = lax.axis_index("s") * nc + lax.axis_index("c")
        base = wid * bpw
        i1 = pltpu.async_copy(stu_hbm.at[pl.ds(base, bpw)], stu_v, sem)
        i2 = pltpu.async_copy(exe_hbm.at[pl.ds(base, bpw)], exe_v, sem)
        i2.wait()
        for c in range(bpw // 16):
            erow_v[pl.ds(c * 16, 16)] = lax.shift_right_logical(
                exe_v[pl.ds(c * 16, 16)], 7)
        c2 = pltpu.async_copy(kd_hbm.at[exe_v], kd_v, sem)
        c3 = pltpu.async_copy(ep_hbm.at[erow_v], ewin_v, sem)
        i1.wait()
        c1 = pltpu.async_copy(se_hbm.at[stu_v], se_v, sem)
        c2.wait()
        pltpu.sync_copy(kd_v, out_kd.at[pl.ds(base, bpw)])
        c3.wait()
        pltpu.sync_copy(ewin_v, out_ew.at[pl.ds(base, bpw)])
        c1.wait()
        pltpu.sync_copy(se_v, out_se.at[pl.ds(base, bpw)])

    return k(stu_id, exe_id, student_emb, k_difficulty, e_pad)


def _tc_body(tt_ref, wkc_ref, bkc_ref, ikp_ref, se_ref, kd_ref, ew_ref,
             exe_ref, w1_ref, b1_ref, w2_ref, b2_ref, w3_ref, b3_ref,
             out_ref):
    x = lax.dot_general(tt_ref[...], wkc_ref[...], (((1,), (1,)), ((), ())),
                        preferred_element_type=jnp.float32)
    x = x + bkc_ref[...]
    ikp = ikp_ref[...]
    col = lax.bitwise_and(exe_ref[...], 127)
    lane = lax.broadcasted_iota(jnp.int32, ew_ref.shape, 1)
    ed = jnp.sum(jnp.where(lane == col, ew_ref[...], 0.0), axis=1,
                 keepdims=True)
    m1 = jnp.max(x, axis=1, keepdims=True)
    x2 = jnp.where(x == m1, -jnp.inf, x)
    m2 = jnp.max(x2, axis=1, keepdims=True)
    merged = jnp.logical_or(x >= m2, ikp != 0.0)
    s = x + ikp
    norm = jnp.maximum(jnp.sqrt(jnp.sum(s * s, axis=1, keepdims=True)), 1e-12)
    ikp2 = jnp.where(merged, s / norm, 0.0)
    stat = jax.nn.sigmoid(se_ref[...])
    esig = jax.nn.sigmoid(ed)
    input_x = esig * (stat - kd_ref[...]) * ikp2
    h = jax.nn.sigmoid(
        lax.dot_general(input_x, jnp.abs(w1_ref[...]), (((1,), (1,)), ((), ())),
                        preferred_element_type=jnp.float32) + b1_ref[...])
    h = jax.nn.sigmoid(
        lax.dot_general(h, jnp.abs(w2_ref[...]), (((1,), (1,)), ((), ())),
                        preferred_element_type=jnp.float32) + b2_ref[...])
    o = jax.nn.sigmoid(
        lax.dot_general(jnp.abs(w3_ref[...]), h, (((1,), (1,)), ((), ())),
                        preferred_element_type=jnp.float32) + b3_ref[0])
    out_ref[...] = o[0]


def _tc_dense(tt, wkc, bkc, ikp, se, kd, ew, exe, w1, b1, w2, b2, w3, b3):
    bb = 512
    grid = _B // bb
    return pl.pallas_call(
        _tc_body,
        grid=(grid,),
        in_specs=[
            pl.BlockSpec((bb, _T), lambda i: (i, 0)),
            pl.BlockSpec((_K, _T), lambda i: (0, 0)),
            pl.BlockSpec((1, _K), lambda i: (0, 0)),
            pl.BlockSpec((bb, _K), lambda i: (i, 0)),
            pl.BlockSpec((bb, _K), lambda i: (i, 0)),
            pl.BlockSpec((bb, _K), lambda i: (i, 0)),
            pl.BlockSpec((bb, _K), lambda i: (i, 0)),
            pl.BlockSpec((bb, 1), lambda i: (i, 0)),
            pl.BlockSpec((512, _K), lambda i: (0, 0)),
            pl.BlockSpec((1, 512), lambda i: (0, 0)),
            pl.BlockSpec((256, 512), lambda i: (0, 0)),
            pl.BlockSpec((1, 256), lambda i: (0, 0)),
            pl.BlockSpec((1, 256), lambda i: (0, 0)),
            pl.BlockSpec(memory_space=pltpu.SMEM),
        ],
        out_specs=pl.BlockSpec((bb,), lambda i: (i,)),
        out_shape=jax.ShapeDtypeStruct((_B,), jnp.float32),
    )(tt, wkc, bkc.reshape(1, _K), ikp, se, kd, ew, exe.reshape(_B, 1),
      w1, b1.reshape(1, 512), w2, b2.reshape(1, 256), w3, b3)


def kernel(stu_id, input_exercise, input_knowledge_point, teacher_text,
           student_emb, k_difficulty, e_difficulty, Wkc, bkc,
           W1, b1, W2, b2, W3, b3):
    n_e = e_difficulty.shape[0]
    ew_rows = (n_e + _K - 1) // _K
    e_pad = jnp.concatenate(
        [e_difficulty.reshape(-1),
         jnp.zeros((ew_rows * _K - n_e,), jnp.float32)]).reshape(ew_rows, _K)
    se, kd, ew = _sc_gather(stu_id, input_exercise,
                            student_emb, k_difficulty, e_pad)
    return _tc_dense(teacher_text, Wkc, bkc, input_knowledge_point,
                     se, kd, ew, input_exercise, W1, b1, W2, b2, W3, b3)
